# initial kernel scaffold (unmeasured)
import jax
import jax.numpy as jnp
from jax import lax
from jax.experimental import pallas as pl
from jax.experimental.pallas import tpu as pltpu

N_DEV = 4


def kernel(x, w_mat, scale_x, scale_w):
    m_glob, k_shard = x.shape
    k_glob, n = w_mat.shape
    m_per = m_glob // N_DEV
    fp8 = jnp.float8_e5m2

    def body(x_ref, w_ref, sx_ref, sw_ref, out_ref,
             xq_ref, recv_ref, send_sems, recv_sems):
        my = lax.axis_index("i")

        xq_ref[...] = x_ref[...].astype(fp8)

        bsem = pltpu.get_barrier_semaphore()
        for k in range(1, N_DEV):
            peer = lax.rem(my + k, N_DEV)
            pl.semaphore_signal(
                bsem, inc=1,
                device_id=(peer,), device_id_type=pl.DeviceIdType.MESH,
            )
        pl.semaphore_wait(bsem, N_DEV - 1)

        sends = []
        for k in range(1, N_DEV):
            dst = lax.rem(my + k, N_DEV)
            rdma = pltpu.make_async_remote_copy(
                src_ref=xq_ref.at[pl.ds(dst * m_per, m_per), :],
                dst_ref=recv_ref.at[k - 1],
                send_sem=send_sems.at[k - 1],
                recv_sem=recv_sems.at[k - 1],
                device_id=(dst,),
                device_id_type=pl.DeviceIdType.MESH,
            )
            rdma.start()
            sends.append(rdma)

        wblk = w_ref[pl.ds(my * k_shard, k_shard), :].astype(fp8)
        out_ref[...] = jnp.dot(
            xq_ref[pl.ds(my * m_per, m_per), :], wblk,
            preferred_element_type=jnp.float32,
        )

        for k in range(1, N_DEV):
            src = lax.rem(my - k + N_DEV, N_DEV)
            sends[k - 1].wait_recv()
            wblk = w_ref[pl.ds(src * k_shard, k_shard), :].astype(fp8)
            out_ref[...] += jnp.dot(
                recv_ref[k - 1], wblk, preferred_element_type=jnp.float32,
            )

        s = sx_ref[0] * sw_ref[0]
        y = out_ref[...] * s
        z = jnp.clip(y, -60.0, 60.0)
        out_ref[...] = y / (1.0 + jnp.exp(-z))

        for rdma in sends:
            rdma.wait_send()

    return pl.pallas_call(
        body,
        out_shape=jax.ShapeDtypeStruct((m_per, n), jnp.float32),
        in_specs=[
            pl.BlockSpec(memory_space=pltpu.VMEM),
            pl.BlockSpec(memory_space=pltpu.VMEM),
            pl.BlockSpec(memory_space=pltpu.SMEM),
            pl.BlockSpec(memory_space=pltpu.SMEM),
        ],
        out_specs=pl.BlockSpec(memory_space=pltpu.VMEM),
        scratch_shapes=[
            pltpu.VMEM((m_glob, k_shard), fp8),
            pltpu.VMEM((N_DEV - 1, m_per, k_shard), fp8),
            pltpu.SemaphoreType.DMA((N_DEV - 1,)),
            pltpu.SemaphoreType.DMA((N_DEV - 1,)),
        ],
        compiler_params=pltpu.CompilerParams(collective_id=0),
    )(x, w_mat, scale_x, scale_w)


# baseline (device time: 59103 ns/iter reference)
import jax
import jax.numpy as jnp
from jax import lax
from jax.experimental import pallas as pl
from jax.experimental.pallas import tpu as pltpu

N_DEV = 4


def kernel(x, w_mat, scale_x, scale_w):
    m_glob, k_shard = x.shape
    k_glob, n = w_mat.shape
    m_per = m_glob // N_DEV
    fp8 = jnp.float8_e5m2

    def body(x_ref, w_ref, sx_ref, sw_ref, out_ref,
             xq_ref, recv_ref, send_sems, recv_sems):
        my = lax.axis_index("i")

        xq_ref[...] = x_ref[...].astype(fp8)

        bsem = pltpu.get_barrier_semaphore()
        for k in range(1, N_DEV):
            peer = lax.rem(my + k, N_DEV)
            pl.semaphore_signal(
                bsem, inc=1,
                device_id=(peer,), device_id_type=pl.DeviceIdType.MESH,
            )
        pl.semaphore_wait(bsem, N_DEV - 1)

        sends = []
        for k in range(1, N_DEV):
            dst = lax.rem(my + k, N_DEV)
            rdma = pltpu.make_async_remote_copy(
                src_ref=xq_ref.at[pl.ds(dst * m_per, m_per), :],
                dst_ref=recv_ref.at[k - 1],
                send_sem=send_sems.at[k - 1],
                recv_sem=recv_sems.at[k - 1],
                device_id=(dst,),
                device_id_type=pl.DeviceIdType.MESH,
            )
            rdma.start()
            sends.append(rdma)

        wblk = w_ref[pl.ds(my * k_shard, k_shard), :].astype(fp8)
        out_ref[...] = jnp.dot(
            xq_ref[pl.ds(my * m_per, m_per), :], wblk,
            preferred_element_type=jnp.float32,
        )

        for k in range(1, N_DEV):
            src = lax.rem(my - k + N_DEV, N_DEV)
            sends[k - 1].wait_recv()
            wblk = w_ref[pl.ds(src * k_shard, k_shard), :].astype(fp8)
            out_ref[...] += jnp.dot(
                recv_ref[k - 1], wblk, preferred_element_type=jnp.float32,
            )

        s = sx_ref[0] * sw_ref[0]
        y = out_ref[...] * s
        z = jnp.clip(y, -60.0, 60.0)
        out_ref[...] = y / (1.0 + jnp.exp(-z))

        for rdma in sends:
            rdma.wait_send()

    return pl.pallas_call(
        body,
        out_shape=jax.ShapeDtypeStruct((m_per, n), jnp.float32),
        in_specs=[
            pl.BlockSpec(memory_space=pltpu.VMEM),
            pl.BlockSpec(memory_space=pltpu.VMEM),
            pl.BlockSpec(memory_space=pltpu.SMEM),
            pl.BlockSpec(memory_space=pltpu.SMEM),
        ],
        out_specs=pl.BlockSpec(memory_space=pltpu.VMEM),
        scratch_shapes=[
            pltpu.VMEM((m_glob, k_shard), fp8),
            pltpu.VMEM((N_DEV - 1, m_per, k_shard), fp8),
            pltpu.SemaphoreType.DMA((N_DEV - 1,)),
            pltpu.SemaphoreType.DMA((N_DEV - 1,)),
        ],
        compiler_params=pltpu.CompilerParams(
            collective_id=0,
            vmem_limit_bytes=100 * 1024 * 1024,
        ),
    )(x, w_mat, scale_x, scale_w)
